# Initial kernel scaffold; baseline (speedup 1.0000x reference)
import jax
import jax.numpy as jnp
from jax.experimental import pallas as pl


def kernel(n_feature, edge_index, attn_l, attn_r, few1, feb1, feg1, febt1, few2, feb2, feg2, febt2, fnw1, fnb1, fng1, fnbt1, fnw2, fnb2, fng2, fnbt2):
    raise NotImplementedError("write your pallas kernel here")


# speedup vs baseline: 4.9172x; 4.9172x over previous
import functools

import jax
import jax.numpy as jnp
from jax import lax
from jax.experimental import pallas as pl
from jax.experimental.pallas import tpu as pltpu
from jax.experimental.pallas import tpu_sc as plsc

N = 10000
E = 160000
D = 256
HID = 4

NPAD = 10240
NT = 1024
ET = 1024
NW = 32
CHUNK = 128
EPAD = 163840
B_CH = EPAD // NW // CHUNK
D_CH = EPAD // 16 // CHUNK
NHALF = 5000
ACC_ROWS = 5120
SLAB = ACC_ROWS // 16


def _precomp_body(h_ref, w_ref, out_ref, mx_ref):
    p = jnp.dot(h_ref[...], w_ref[...], preferred_element_type=jnp.float32)
    out_ref[...] = p
    mx_ref[...] = jnp.max(p, axis=0).reshape(1, 1, 16)


def _node_precompute(h_pad, w_pre):
    grid = (NPAD // NT,)
    return pl.pallas_call(
        _precomp_body,
        grid=grid,
        in_specs=[
            pl.BlockSpec((NT, D), lambda i: (i, 0)),
            pl.BlockSpec((D, 16), lambda i: (0, 0)),
        ],
        out_specs=[
            pl.BlockSpec((NT, 16), lambda i: (i, 0)),
            pl.BlockSpec((1, 1, 16), lambda i: (i, 0, 0)),
        ],
        out_shape=[
            jax.ShapeDtypeStruct((NPAD, 16), jnp.float32),
            jax.ShapeDtypeStruct((NPAD // NT, 1, 16), jnp.float32),
        ],
    )(h_pad, w_pre)


def _gather_body(pre_hbm, src_hbm, dst_hbm, gs_hbm, gd_hbm, idx_v, rows_v, sem):
    wid = lax.axis_index("s") * 2 + lax.axis_index("c")
    base = wid * (B_CH * CHUNK)

    def chunk(j, _):
        off = base + j * CHUNK
        pltpu.sync_copy(src_hbm.at[pl.ds(off, CHUNK)], idx_v)
        pltpu.async_copy(pre_hbm.at[idx_v], rows_v, sem).wait()
        pltpu.sync_copy(rows_v, gs_hbm.at[pl.ds(off, CHUNK)])
        pltpu.sync_copy(dst_hbm.at[pl.ds(off, CHUNK)], idx_v)
        pltpu.async_copy(pre_hbm.at[idx_v], rows_v, sem).wait()
        pltpu.sync_copy(rows_v, gd_hbm.at[pl.ds(off, CHUNK)])
        return 0

    lax.fori_loop(0, B_CH, chunk, 0)


_edge_gather = functools.partial(
    pl.kernel,
    mesh=plsc.VectorSubcoreMesh(core_axis_name="c", subcore_axis_name="s"),
    compiler_params=pltpu.CompilerParams(use_tc_tiling_on_sc=False),
    out_type=[
        jax.ShapeDtypeStruct((EPAD, 16), jnp.float32),
        jax.ShapeDtypeStruct((EPAD, 16), jnp.float32),
    ],
    scratch_types=[
        pltpu.VMEM((CHUNK,), jnp.int32),
        pltpu.VMEM((CHUNK, 16), jnp.float32),
        pltpu.SemaphoreType.DMA,
    ],
)(_gather_body)


def _edge_body(gs_ref, gd_ref, g_ref, c1_ref, b1_ref, w2_ref, b2_ref,
               wn1_ref, c3_ref, b3_ref, w4_ref, b4_ref, w_ref, e_ref):
    gs = gs_ref[...]
    gd = gd_ref[...]
    u = gs[:, 0:1] + gd[:, 1:2]
    u = jnp.where(u >= 0, u, 0.2 * u)
    e = jnp.exp(u - g_ref[0, 0])
    x1 = gs[:, 2:6] + gd[:, 6:10]
    x1 = c1_ref[...] * x1 + b1_ref[...]
    x1 = jnp.where(x1 >= 0, x1, 0.01 * x1)
    x2 = jnp.dot(x1, w2_ref[...], preferred_element_type=jnp.float32)
    x2 = jnp.maximum(x2 + b2_ref[...], 0.0)
    t = gs[:, 10:14] + jnp.dot(x2, wn1_ref[...],
                               preferred_element_type=jnp.float32)
    x3 = c3_ref[...] * t + b3_ref[...]
    x3 = jnp.where(x3 >= 0, x3, 0.01 * x3)
    msg = jnp.dot(x3, w4_ref[...], preferred_element_type=jnp.float32)
    msg = jnp.maximum(msg + b4_ref[...], 0.0)
    w_ref[...] = msg * e
    e_ref[...] = e


def _edge_dense(gs, gd, g, c1, b1, w2, b2, wn1, c3, b3, w4, b4):
    grid = (EPAD // ET,)
    full = lambda r, c: pl.BlockSpec((r, c), lambda i: (0, 0))
    return pl.pallas_call(
        _edge_body,
        grid=grid,
        in_specs=[
            pl.BlockSpec((ET, 16), lambda i: (i, 0)),
            pl.BlockSpec((ET, 16), lambda i: (i, 0)),
            full(1, 1), full(1, HID), full(1, HID), full(HID, D), full(1, D),
            full(D, HID), full(1, HID), full(1, HID), full(HID, D), full(1, D),
        ],
        out_specs=[
            pl.BlockSpec((ET, D), lambda i: (i, 0)),
            pl.BlockSpec((ET, 1), lambda i: (i, 0)),
        ],
        out_shape=[
            jax.ShapeDtypeStruct((EPAD, D), jnp.float32),
            jax.ShapeDtypeStruct((EPAD, 1), jnp.float32),
        ],
    )(gs, gd, g, c1, b1, w2, b2, wn1, c3, b3, w4, b4)


def _scatter_body(w_hbm, e_hbm, dst3_hbm, zrows_hbm, z1_hbm, acc_out, s_out,
                  idx_v, il_v, w_v, e_v, acc_sh, s_sh, sem):
    c = lax.axis_index("c")
    s = lax.axis_index("s")
    nbase = c * NHALF
    pltpu.sync_copy(zrows_hbm.at[pl.ds(s * SLAB, SLAB)],
                    acc_sh.at[pl.ds(s * SLAB, SLAB)])
    pltpu.sync_copy(z1_hbm.at[pl.ds(s * SLAB, SLAB)],
                    s_sh.at[pl.ds(s * SLAB, SLAB)])
    plsc.subcore_barrier()

    ebase = s * (D_CH * CHUNK)

    def chunk(j, _):
        off = ebase + j * CHUNK
        pltpu.sync_copy(dst3_hbm.at[s, j], idx_v)
        for k in range(CHUNK // 16):
            v = idx_v[pl.ds(k * 16, 16)]
            il = v - nbase
            bad = (il < 0) | (il >= NHALF)
            il_v[pl.ds(k * 16, 16)] = jnp.where(bad, NHALF, il)
        pltpu.sync_copy(w_hbm.at[pl.ds(off, CHUNK)], w_v)
        pltpu.sync_copy(w_v, acc_sh.at[il_v], add=True)
        pltpu.sync_copy(e_hbm.at[pl.ds(off, CHUNK)], e_v)
        pltpu.sync_copy(e_v, s_sh.at[il_v], add=True)
        return 0

    lax.fori_loop(0, D_CH, chunk, 0)
    plsc.subcore_barrier()
    pltpu.sync_copy(acc_sh.at[pl.ds(s * SLAB, SLAB)],
                    acc_out.at[c, pl.ds(s * SLAB, SLAB)])
    pltpu.sync_copy(s_sh.at[pl.ds(s * SLAB, SLAB)],
                    s_out.at[c, pl.ds(s * SLAB, SLAB)])


_edge_scatter = functools.partial(
    pl.kernel,
    mesh=plsc.VectorSubcoreMesh(core_axis_name="c", subcore_axis_name="s"),
    compiler_params=pltpu.CompilerParams(use_tc_tiling_on_sc=False),
    out_type=[
        jax.ShapeDtypeStruct((2, ACC_ROWS, D), jnp.float32),
        jax.ShapeDtypeStruct((2, ACC_ROWS), jnp.float32),
    ],
    scratch_types=[
        pltpu.VMEM((CHUNK,), jnp.int32),
        pltpu.VMEM((CHUNK,), jnp.int32),
        pltpu.VMEM((CHUNK, D), jnp.float32),
        pltpu.VMEM((CHUNK,), jnp.float32),
        pltpu.VMEM_SHARED((ACC_ROWS, D), jnp.float32),
        pltpu.VMEM_SHARED((ACC_ROWS,), jnp.float32),
        pltpu.SemaphoreType.DMA,
    ],
)(_scatter_body)


def _final_body(acc_ref, s_ref, h_ref, o_ref):
    o_ref[...] = acc_ref[...] / (s_ref[...] + 1e-16) + h_ref[...]


def _finalize(acc, s3, h2):
    grid = (2, ACC_ROWS // 512)
    return pl.pallas_call(
        _final_body,
        grid=grid,
        in_specs=[
            pl.BlockSpec((1, 512, D), lambda i, j: (i, j, 0)),
            pl.BlockSpec((1, 512, 1), lambda i, j: (i, j, 0)),
            pl.BlockSpec((1, 512, D), lambda i, j: (i, j, 0)),
        ],
        out_specs=pl.BlockSpec((1, 512, D), lambda i, j: (i, j, 0)),
        out_shape=jax.ShapeDtypeStruct((2, ACC_ROWS, D), jnp.float32),
    )(acc, s3, h2)


def kernel(n_feature, edge_index, attn_l, attn_r, few1, feb1, feg1, febt1,
           few2, feb2, feg2, febt2, fnw1, fnb1, fng1, fnbt1, fnw2, fnb2,
           fng2, fnbt2):
    rs = 1.0 / jnp.sqrt(jnp.float32(1.0 + 1e-5))

    w_pre = jnp.concatenate([
        attn_l.T, attn_r.T, few1[:D], few1[D:], fnw1[:D],
        jnp.zeros((D, 2), jnp.float32),
    ], axis=1)
    h_pad = jnp.pad(n_feature, ((0, NPAD - N), (0, 0)))
    precomp, mx = _node_precompute(h_pad, w_pre)

    g = jnp.maximum(jnp.max(mx[:, 0, 0]) + jnp.max(mx[:, 0, 1]), 0.0)
    g = g.reshape(1, 1)

    src = edge_index[0]
    dst = edge_index[1]
    srcp = jnp.concatenate([src, jnp.zeros((EPAD - E,), jnp.int32)])
    dstp = jnp.concatenate([dst, jnp.full((EPAD - E,), N, jnp.int32)])

    gs, gd = _edge_gather(precomp, srcp, dstp)

    c1 = (feg1 * rs).reshape(1, HID)
    b1 = (feg1 * rs * feb1 + febt1).reshape(1, HID)
    s2 = feg2 * rs
    w2 = few2 * s2[None, :]
    b2 = (s2 * feb2 + febt2).reshape(1, D)
    c3 = (fng1 * rs).reshape(1, HID)
    b3 = (fng1 * rs * fnb1 + fnbt1).reshape(1, HID)
    s4 = fng2 * rs
    w4 = fnw2 * s4[None, :]
    b4 = (s4 * fnb2 + fnbt2).reshape(1, D)

    w, e = _edge_dense(gs, gd, g, c1, b1, w2, b2, fnw1[D:], c3, b3, w4, b4)

    dst3 = dstp.reshape(16, D_CH, CHUNK)
    zrows = jnp.zeros((ACC_ROWS, D), jnp.float32)
    z1 = jnp.zeros((ACC_ROWS,), jnp.float32)
    acc, seg = _edge_scatter(w, e.reshape(EPAD), dst3, zrows, z1)

    h2 = jnp.stack([
        jnp.pad(n_feature[:NHALF], ((0, ACC_ROWS - NHALF), (0, 0))),
        jnp.pad(n_feature[NHALF:], ((0, ACC_ROWS - NHALF), (0, 0))),
    ])
    out2 = _finalize(acc, seg.reshape(2, ACC_ROWS, 1), h2)
    return jnp.concatenate([out2[0, :NHALF], out2[1, :NHALF]], axis=0)


# speedup vs baseline: 5.8520x; 1.1901x over previous
import functools

import jax
import jax.numpy as jnp
from jax import lax
from jax.experimental import pallas as pl
from jax.experimental.pallas import tpu as pltpu
from jax.experimental.pallas import tpu_sc as plsc

N = 10000
E = 160000
D = 256
HID = 4

NT = 400
ET = 1024
NW = 32
CHUNK = 128
EPAD = 163840
B_CH = EPAD // NW // CHUNK
CHD = 64
D_CH = EPAD // 16 // CHD
NHALF = 5120
ACC_ROWS = 5248
SLAB_I = ACC_ROWS // 16
SLAB_O = NHALF // 16


def _precomp_body(h_ref, w_ref, out_ref, mx_ref):
    p = jnp.dot(h_ref[...], w_ref[...], preferred_element_type=jnp.float32)
    out_ref[...] = p
    mx_ref[...] = jnp.max(p, axis=0).reshape(1, 1, 16)


def _node_precompute(h, w_pre):
    grid = (N // NT,)
    return pl.pallas_call(
        _precomp_body,
        grid=grid,
        in_specs=[
            pl.BlockSpec((NT, D), lambda i: (i, 0)),
            pl.BlockSpec((D, 16), lambda i: (0, 0)),
        ],
        out_specs=[
            pl.BlockSpec((NT, 16), lambda i: (i, 0)),
            pl.BlockSpec((1, 1, 16), lambda i: (i, 0, 0)),
        ],
        out_shape=[
            jax.ShapeDtypeStruct((N, 16), jnp.float32),
            jax.ShapeDtypeStruct((N // NT, 1, 16), jnp.float32),
        ],
    )(h, w_pre)


def _gather_body(pre_hbm, src_hbm, dst_hbm, gs_hbm, gd_hbm,
                 is0, is1, id0, id1, rs0, rs1, rd0, rd1,
                 ss0, ss1, sd0, sd1):
    wid = lax.axis_index("s") * 2 + lax.axis_index("c")
    base = wid * (B_CH * CHUNK)
    isv = (is0, is1)
    idv = (id0, id1)
    rsv = (rs0, rs1)
    rdv = (rd0, rd1)
    ssv = (ss0, ss1)
    sdv = (sd0, sd1)

    pltpu.sync_copy(src_hbm.at[pl.ds(base, CHUNK)], is0)
    pltpu.sync_copy(dst_hbm.at[pl.ds(base, CHUNK)], id0)
    pltpu.async_copy(pre_hbm.at[is0], rs0, ss0)
    pltpu.async_copy(pre_hbm.at[id0], rd0, sd0)

    def pair(i, _):
        for b in (0, 1):
            j = 2 * i + b
            nxt = 1 - b
            @pl.when(j + 1 < B_CH)
            def _():
                noff = base + (j + 1) * CHUNK
                pltpu.sync_copy(src_hbm.at[pl.ds(noff, CHUNK)], isv[nxt])
                pltpu.sync_copy(dst_hbm.at[pl.ds(noff, CHUNK)], idv[nxt])
                pltpu.async_copy(pre_hbm.at[isv[nxt]], rsv[nxt], ssv[nxt])
                pltpu.async_copy(pre_hbm.at[idv[nxt]], rdv[nxt], sdv[nxt])
            off = base + j * CHUNK
            pltpu.make_async_copy(pre_hbm.at[isv[b]], rsv[b], ssv[b]).wait()
            pltpu.sync_copy(rsv[b], gs_hbm.at[pl.ds(off, CHUNK)])
            pltpu.make_async_copy(pre_hbm.at[idv[b]], rdv[b], sdv[b]).wait()
            pltpu.sync_copy(rdv[b], gd_hbm.at[pl.ds(off, CHUNK)])
        return 0

    lax.fori_loop(0, B_CH // 2, pair, 0)


_edge_gather = functools.partial(
    pl.kernel,
    mesh=plsc.VectorSubcoreMesh(core_axis_name="c", subcore_axis_name="s"),
    compiler_params=pltpu.CompilerParams(use_tc_tiling_on_sc=False),
    out_type=[
        jax.ShapeDtypeStruct((EPAD, 16), jnp.float32),
        jax.ShapeDtypeStruct((EPAD, 16), jnp.float32),
    ],
    scratch_types=[
        pltpu.VMEM((CHUNK,), jnp.int32),
        pltpu.VMEM((CHUNK,), jnp.int32),
        pltpu.VMEM((CHUNK,), jnp.int32),
        pltpu.VMEM((CHUNK,), jnp.int32),
        pltpu.VMEM((CHUNK, 16), jnp.float32),
        pltpu.VMEM((CHUNK, 16), jnp.float32),
        pltpu.VMEM((CHUNK, 16), jnp.float32),
        pltpu.VMEM((CHUNK, 16), jnp.float32),
        pltpu.SemaphoreType.DMA,
        pltpu.SemaphoreType.DMA,
        pltpu.SemaphoreType.DMA,
        pltpu.SemaphoreType.DMA,
    ],
)(_gather_body)


def _edge_body(gs_ref, gd_ref, g_ref, c1_ref, b1_ref, w2_ref, b2_ref,
               wn1_ref, c3_ref, b3_ref, w4_ref, b4_ref, w_ref, e_ref):
    gs = gs_ref[...]
    gd = gd_ref[...]
    u = gs[:, 0:1] + gd[:, 1:2]
    u = jnp.where(u >= 0, u, 0.2 * u)
    e = jnp.exp(u - g_ref[0, 0])
    x1 = gs[:, 2:6] + gd[:, 6:10]
    x1 = c1_ref[...] * x1 + b1_ref[...]
    x1 = jnp.where(x1 >= 0, x1, 0.01 * x1)
    x2 = jnp.dot(x1, w2_ref[...], preferred_element_type=jnp.float32)
    x2 = jnp.maximum(x2 + b2_ref[...], 0.0)
    t = gs[:, 10:14] + jnp.dot(x2, wn1_ref[...],
                               preferred_element_type=jnp.float32)
    x3 = c3_ref[...] * t + b3_ref[...]
    x3 = jnp.where(x3 >= 0, x3, 0.01 * x3)
    msg = jnp.dot(x3, w4_ref[...], preferred_element_type=jnp.float32)
    msg = jnp.maximum(msg + b4_ref[...], 0.0)
    w_ref[...] = msg * e
    e_ref[...] = e


def _edge_dense(gs, gd, g, c1, b1, w2, b2, wn1, c3, b3, w4, b4):
    grid = (EPAD // ET,)
    full = lambda r, c: pl.BlockSpec((r, c), lambda i: (0, 0))
    return pl.pallas_call(
        _edge_body,
        grid=grid,
        in_specs=[
            pl.BlockSpec((ET, 16), lambda i: (i, 0)),
            pl.BlockSpec((ET, 16), lambda i: (i, 0)),
            full(1, 1), full(1, HID), full(1, HID), full(HID, D), full(1, D),
            full(D, HID), full(1, HID), full(1, HID), full(HID, D), full(1, D),
        ],
        out_specs=[
            pl.BlockSpec((ET, D), lambda i: (i, 0)),
            pl.BlockSpec((ET, 1), lambda i: (i, 0)),
        ],
        out_shape=[
            jax.ShapeDtypeStruct((EPAD, D), jnp.float32),
            jax.ShapeDtypeStruct((EPAD, 1), jnp.float32),
        ],
    )(gs, gd, g, c1, b1, w2, b2, wn1, c3, b3, w4, b4)


def _scatter_body(w_hbm, e_hbm, dst3_hbm, zrows_hbm, z1_hbm, acc_out, s_out,
                  idx_v, il0, il1, w0, w1, e0, e1, acc_sh, s_sh,
                  sw0, sw1, se0, se1):
    c = lax.axis_index("c")
    s = lax.axis_index("s")
    nbase = c * NHALF
    ilv = (il0, il1)
    wv = (w0, w1)
    ev = (e0, e1)
    swv = (sw0, sw1)
    sev = (se0, se1)

    pltpu.sync_copy(zrows_hbm.at[pl.ds(s * SLAB_I, SLAB_I)],
                    acc_sh.at[pl.ds(s * SLAB_I, SLAB_I)])
    pltpu.sync_copy(z1_hbm.at[pl.ds(s * SLAB_I, SLAB_I)],
                    s_sh.at[pl.ds(s * SLAB_I, SLAB_I)])
    plsc.subcore_barrier()

    ebase = s * (D_CH * CHD)

    def load_il(j, b):
        pltpu.sync_copy(dst3_hbm.at[s, j], idx_v)
        for k in range(CHD // 16):
            v = idx_v[pl.ds(k * 16, 16)]
            il = v - nbase
            bad = (il < 0) | (il >= NHALF)
            ilv[b][pl.ds(k * 16, 16)] = jnp.where(bad, NHALF, il)

    pltpu.async_copy(w_hbm.at[pl.ds(ebase, CHD)], w0, sw0)
    pltpu.async_copy(e_hbm.at[pl.ds(ebase, CHD)], e0, se0)
    load_il(0, 0)

    def pair(i, _):
        for b in (0, 1):
            j = 2 * i + b
            nxt = 1 - b
            @pl.when(j + 1 < D_CH)
            def _():
                noff = ebase + (j + 1) * CHD
                pltpu.async_copy(w_hbm.at[pl.ds(noff, CHD)], wv[nxt],
                                 swv[nxt])
                pltpu.async_copy(e_hbm.at[pl.ds(noff, CHD)], ev[nxt],
                                 sev[nxt])
                load_il(j + 1, nxt)
            off = ebase + j * CHD
            pltpu.make_async_copy(w_hbm.at[pl.ds(off, CHD)], wv[b],
                                  swv[b]).wait()
            pltpu.sync_copy(wv[b], acc_sh.at[ilv[b]], add=True)
            pltpu.make_async_copy(e_hbm.at[pl.ds(off, CHD)], ev[b],
                                  sev[b]).wait()
            pltpu.sync_copy(ev[b], s_sh.at[ilv[b]], add=True)
        return 0

    lax.fori_loop(0, D_CH // 2, pair, 0)
    plsc.subcore_barrier()
    pltpu.sync_copy(acc_sh.at[pl.ds(s * SLAB_O, SLAB_O)],
                    acc_out.at[c, pl.ds(s * SLAB_O, SLAB_O)])
    pltpu.sync_copy(s_sh.at[pl.ds(s * SLAB_O, SLAB_O)],
                    s_out.at[c, pl.ds(s * SLAB_O, SLAB_O)])


_edge_scatter = functools.partial(
    pl.kernel,
    mesh=plsc.VectorSubcoreMesh(core_axis_name="c", subcore_axis_name="s"),
    compiler_params=pltpu.CompilerParams(use_tc_tiling_on_sc=False),
    out_type=[
        jax.ShapeDtypeStruct((2, NHALF, D), jnp.float32),
        jax.ShapeDtypeStruct((2, NHALF), jnp.float32),
    ],
    scratch_types=[
        pltpu.VMEM((CHD,), jnp.int32),
        pltpu.VMEM((CHD,), jnp.int32),
        pltpu.VMEM((CHD,), jnp.int32),
        pltpu.VMEM((CHD, D), jnp.float32),
        pltpu.VMEM((CHD, D), jnp.float32),
        pltpu.VMEM((CHD,), jnp.float32),
        pltpu.VMEM((CHD,), jnp.float32),
        pltpu.VMEM_SHARED((ACC_ROWS, D), jnp.float32),
        pltpu.VMEM_SHARED((ACC_ROWS,), jnp.float32),
        pltpu.SemaphoreType.DMA,
        pltpu.SemaphoreType.DMA,
        pltpu.SemaphoreType.DMA,
        pltpu.SemaphoreType.DMA,
    ],
)(_scatter_body)


def _final_body(acc_ref, s_ref, h_ref, o_ref):
    o_ref[...] = acc_ref[...] / (s_ref[...] + 1e-16) + h_ref[...]


def _finalize(acc_flat, s_flat, h):
    grid = (N // NT,)
    return pl.pallas_call(
        _final_body,
        grid=grid,
        in_specs=[
            pl.BlockSpec((NT, D), lambda i: (i, 0)),
            pl.BlockSpec((NT, 1), lambda i: (i, 0)),
            pl.BlockSpec((NT, D), lambda i: (i, 0)),
        ],
        out_specs=pl.BlockSpec((NT, D), lambda i: (i, 0)),
        out_shape=jax.ShapeDtypeStruct((N, D), jnp.float32),
    )(acc_flat, s_flat, h)


def kernel(n_feature, edge_index, attn_l, attn_r, few1, feb1, feg1, febt1,
           few2, feb2, feg2, febt2, fnw1, fnb1, fng1, fnbt1, fnw2, fnb2,
           fng2, fnbt2):
    rs = 1.0 / jnp.sqrt(jnp.float32(1.0 + 1e-5))

    w_pre = jnp.concatenate([
        attn_l.T, attn_r.T, few1[:D], few1[D:], fnw1[:D],
        jnp.zeros((D, 2), jnp.float32),
    ], axis=1)
    precomp, mx = _node_precompute(n_feature, w_pre)

    g = jnp.maximum(jnp.max(mx[:, 0, 0]) + jnp.max(mx[:, 0, 1]), 0.0)
    g = g.reshape(1, 1)

    src = edge_index[0]
    dst = edge_index[1]
    srcp = jnp.concatenate([src, jnp.zeros((EPAD - E,), jnp.int32)])
    dstp = jnp.concatenate([dst, jnp.zeros((EPAD - E,), jnp.int32)])
    dsts = jnp.concatenate([dst, jnp.full((EPAD - E,), 2 * NHALF, jnp.int32)])

    gs, gd = _edge_gather(precomp, srcp, dstp)

    c1 = (feg1 * rs).reshape(1, HID)
    b1 = (feg1 * rs * feb1 + febt1).reshape(1, HID)
    s2 = feg2 * rs
    w2 = few2 * s2[None, :]
    b2 = (s2 * feb2 + febt2).reshape(1, D)
    c3 = (fng1 * rs).reshape(1, HID)
    b3 = (fng1 * rs * fnb1 + fnbt1).reshape(1, HID)
    s4 = fng2 * rs
    w4 = fnw2 * s4[None, :]
    b4 = (s4 * fnb2 + fnbt2).reshape(1, D)

    w, e = _edge_dense(gs, gd, g, c1, b1, w2, b2, fnw1[D:], c3, b3, w4, b4)

    dst3 = dsts.reshape(16, D_CH, CHD)
    zrows = jnp.zeros((ACC_ROWS, D), jnp.float32)
    z1 = jnp.zeros((ACC_ROWS,), jnp.float32)
    acc, seg = _edge_scatter(w, e.reshape(EPAD), dst3, zrows, z1)

    acc_flat = acc.reshape(2 * NHALF, D)
    s_flat = seg.reshape(2 * NHALF, 1)
    return _finalize(acc_flat, s_flat, n_feature)


# speedup vs baseline: 6.7824x; 1.1590x over previous
import functools

import jax
import jax.numpy as jnp
from jax import lax
from jax.experimental import pallas as pl
from jax.experimental.pallas import tpu as pltpu
from jax.experimental.pallas import tpu_sc as plsc

N = 10000
E = 160000
D = 256
HID = 4

NT = 400
ET = 1024
NW = 32
CHUNK = 128
EPAD = 163840
B_CH = EPAD // NW // CHUNK
CHD = 64
D_CH = EPAD // 16 // CHD
NHALF = 5120
ACC_ROWS = 5248
SLAB_I = ACC_ROWS // 16
SLAB_O = NHALF // 16


def _precomp_body(h_ref, w_ref, out_ref, mx_ref):
    p = jnp.dot(h_ref[...], w_ref[...], preferred_element_type=jnp.float32)
    out_ref[...] = p
    mx_ref[...] = jnp.max(p, axis=0).reshape(1, 1, 16)


def _node_precompute(h, w_pre):
    grid = (N // NT,)
    return pl.pallas_call(
        _precomp_body,
        grid=grid,
        in_specs=[
            pl.BlockSpec((NT, D), lambda i: (i, 0)),
            pl.BlockSpec((D, 16), lambda i: (0, 0)),
        ],
        out_specs=[
            pl.BlockSpec((NT, 16), lambda i: (i, 0)),
            pl.BlockSpec((1, 1, 16), lambda i: (i, 0, 0)),
        ],
        out_shape=[
            jax.ShapeDtypeStruct((N, 16), jnp.float32),
            jax.ShapeDtypeStruct((N // NT, 1, 16), jnp.float32),
        ],
    )(h, w_pre)


def _gather_body(pre_hbm, src_hbm, dst_hbm, gs_hbm, gd_hbm,
                 is0, is1, id0, id1, rs0, rs1, rd0, rd1,
                 ss0, ss1, sd0, sd1):
    wid = lax.axis_index("s") * 2 + lax.axis_index("c")
    base = wid * (B_CH * CHUNK)
    isv = (is0, is1)
    idv = (id0, id1)
    rsv = (rs0, rs1)
    rdv = (rd0, rd1)
    ssv = (ss0, ss1)
    sdv = (sd0, sd1)

    pltpu.sync_copy(src_hbm.at[pl.ds(base, CHUNK)], is0)
    pltpu.sync_copy(dst_hbm.at[pl.ds(base, CHUNK)], id0)
    pltpu.async_copy(pre_hbm.at[is0], rs0, ss0)
    pltpu.async_copy(pre_hbm.at[id0], rd0, sd0)

    def pair(i, _):
        for b in (0, 1):
            j = 2 * i + b
            nxt = 1 - b
            @pl.when(j + 1 < B_CH)
            def _():
                noff = base + (j + 1) * CHUNK
                pltpu.sync_copy(src_hbm.at[pl.ds(noff, CHUNK)], isv[nxt])
                pltpu.sync_copy(dst_hbm.at[pl.ds(noff, CHUNK)], idv[nxt])
                pltpu.async_copy(pre_hbm.at[isv[nxt]], rsv[nxt], ssv[nxt])
                pltpu.async_copy(pre_hbm.at[idv[nxt]], rdv[nxt], sdv[nxt])
            off = base + j * CHUNK
            pltpu.make_async_copy(pre_hbm.at[isv[b]], rsv[b], ssv[b]).wait()
            pltpu.sync_copy(rsv[b], gs_hbm.at[pl.ds(off, CHUNK)])
            pltpu.make_async_copy(pre_hbm.at[idv[b]], rdv[b], sdv[b]).wait()
            pltpu.sync_copy(rdv[b], gd_hbm.at[pl.ds(off, CHUNK)])
        return 0

    lax.fori_loop(0, B_CH // 2, pair, 0)


_edge_gather = functools.partial(
    pl.kernel,
    mesh=plsc.VectorSubcoreMesh(core_axis_name="c", subcore_axis_name="s"),
    compiler_params=pltpu.CompilerParams(use_tc_tiling_on_sc=False),
    out_type=[
        jax.ShapeDtypeStruct((EPAD, 16), jnp.float32),
        jax.ShapeDtypeStruct((EPAD, 16), jnp.float32),
    ],
    scratch_types=[
        pltpu.VMEM((CHUNK,), jnp.int32),
        pltpu.VMEM((CHUNK,), jnp.int32),
        pltpu.VMEM((CHUNK,), jnp.int32),
        pltpu.VMEM((CHUNK,), jnp.int32),
        pltpu.VMEM((CHUNK, 16), jnp.float32),
        pltpu.VMEM((CHUNK, 16), jnp.float32),
        pltpu.VMEM((CHUNK, 16), jnp.float32),
        pltpu.VMEM((CHUNK, 16), jnp.float32),
        pltpu.SemaphoreType.DMA,
        pltpu.SemaphoreType.DMA,
        pltpu.SemaphoreType.DMA,
        pltpu.SemaphoreType.DMA,
    ],
)(_gather_body)


def _edge_body(gs_ref, gd_ref, g_ref, c1_ref, b1_ref, w2_ref, b2_ref,
               wn1_ref, c3_ref, b3_ref, w4_ref, b4_ref, w_ref, e_ref):
    gs = gs_ref[...]
    gd = gd_ref[...]
    u = gs[:, 0:1] + gd[:, 1:2]
    u = jnp.where(u >= 0, u, 0.2 * u)
    e = jnp.exp(u - g_ref[0, 0])
    x1 = gs[:, 2:6] + gd[:, 6:10]
    x1 = c1_ref[...] * x1 + b1_ref[...]
    x1 = jnp.where(x1 >= 0, x1, 0.01 * x1)
    x2 = jnp.dot(x1, w2_ref[...], preferred_element_type=jnp.float32)
    x2 = jnp.maximum(x2 + b2_ref[...], 0.0)
    t = gs[:, 10:14] + jnp.dot(x2, wn1_ref[...],
                               preferred_element_type=jnp.float32)
    x3 = c3_ref[...] * t + b3_ref[...]
    x3 = jnp.where(x3 >= 0, x3, 0.01 * x3)
    msg = jnp.dot(x3, w4_ref[...], preferred_element_type=jnp.float32)
    msg = jnp.maximum(msg + b4_ref[...], 0.0)
    w_ref[...] = (msg * e).reshape(2 * ET, 128)
    e_ref[...] = e


def _edge_dense(gs, gd, g, c1, b1, w2, b2, wn1, c3, b3, w4, b4):
    grid = (EPAD // ET,)
    full = lambda r, c: pl.BlockSpec((r, c), lambda i: (0, 0))
    return pl.pallas_call(
        _edge_body,
        grid=grid,
        in_specs=[
            pl.BlockSpec((ET, 16), lambda i: (i, 0)),
            pl.BlockSpec((ET, 16), lambda i: (i, 0)),
            full(1, 1), full(1, HID), full(1, HID), full(HID, D), full(1, D),
            full(D, HID), full(1, HID), full(1, HID), full(HID, D), full(1, D),
        ],
        out_specs=[
            pl.BlockSpec((2 * ET, 128), lambda i: (i, 0)),
            pl.BlockSpec((ET, 1), lambda i: (i, 0)),
        ],
        out_shape=[
            jax.ShapeDtypeStruct((2 * EPAD, 128), jnp.float32),
            jax.ShapeDtypeStruct((EPAD, 1), jnp.float32),
        ],
    )(gs, gd, g, c1, b1, w2, b2, wn1, c3, b3, w4, b4)


def _scatter_body(w_hbm, e_hbm, dst3_hbm, zrows_hbm, z1_hbm, acc_out, s_out,
                  idx_v, il0, il1, il2a, il2b, w0, w1, e0, e1, acc_sh, s_sh,
                  sw0, sw1, se0, se1):
    c = lax.axis_index("c")
    s = lax.axis_index("s")
    nbase = c * NHALF
    ilv = (il0, il1)
    il2v = (il2a, il2b)
    wv = (w0, w1)
    ev = (e0, e1)
    swv = (sw0, sw1)
    sev = (se0, se1)

    pltpu.sync_copy(zrows_hbm.at[pl.ds(s * (2 * SLAB_I), 2 * SLAB_I)],
                    acc_sh.at[pl.ds(s * (2 * SLAB_I), 2 * SLAB_I)])
    pltpu.sync_copy(z1_hbm.at[pl.ds(s * SLAB_I, SLAB_I)],
                    s_sh.at[pl.ds(s * SLAB_I, SLAB_I)])
    plsc.subcore_barrier()

    ebase = s * (D_CH * CHD)

    lane = lax.iota(jnp.int32, 16)

    def load_il(j, b):
        pltpu.sync_copy(dst3_hbm.at[s, j], idx_v)
        for k in range(CHD // 16):
            v = idx_v[pl.ds(k * 16, 16)]
            il = v - nbase
            bad = (il < 0) | (il >= NHALF)
            ilv[b][pl.ds(k * 16, 16)] = jnp.where(bad, NHALF, il)
        for t in range(2 * CHD // 16):
            src = plsc.load_gather(ilv[b], [t * 8 + (lane >> 1)])
            il2v[b][pl.ds(t * 16, 16)] = 2 * src + (lane & 1)

    pltpu.async_copy(w_hbm.at[pl.ds(2 * ebase, 2 * CHD)], w0, sw0)
    pltpu.async_copy(e_hbm.at[pl.ds(ebase, CHD)], e0, se0)
    load_il(0, 0)

    def pair(i, _):
        for b in (0, 1):
            j = 2 * i + b
            nxt = 1 - b
            @pl.when(j + 1 < D_CH)
            def _():
                noff = ebase + (j + 1) * CHD
                pltpu.async_copy(w_hbm.at[pl.ds(2 * noff, 2 * CHD)], wv[nxt],
                                 swv[nxt])
                pltpu.async_copy(e_hbm.at[pl.ds(noff, CHD)], ev[nxt],
                                 sev[nxt])
                load_il(j + 1, nxt)
            off = ebase + j * CHD
            pltpu.make_async_copy(w_hbm.at[pl.ds(2 * off, 2 * CHD)], wv[b],
                                  swv[b]).wait()
            pltpu.sync_copy(wv[b], acc_sh.at[il2v[b]], add=True)
            pltpu.make_async_copy(e_hbm.at[pl.ds(off, CHD)], ev[b],
                                  sev[b]).wait()
            pltpu.sync_copy(ev[b], s_sh.at[ilv[b]], add=True)
        return 0

    lax.fori_loop(0, D_CH // 2, pair, 0)
    plsc.subcore_barrier()
    pltpu.sync_copy(acc_sh.at[pl.ds(s * (2 * SLAB_O), 2 * SLAB_O)],
                    acc_out.at[c, pl.ds(s * (2 * SLAB_O), 2 * SLAB_O)])
    pltpu.sync_copy(s_sh.at[pl.ds(s * SLAB_O, SLAB_O)],
                    s_out.at[c, pl.ds(s * SLAB_O, SLAB_O)])


_edge_scatter = functools.partial(
    pl.kernel,
    mesh=plsc.VectorSubcoreMesh(core_axis_name="c", subcore_axis_name="s"),
    compiler_params=pltpu.CompilerParams(use_tc_tiling_on_sc=False,
                                         needs_layout_passes=False),
    out_type=[
        jax.ShapeDtypeStruct((2, 2 * NHALF, 128), jnp.float32),
        jax.ShapeDtypeStruct((2, NHALF), jnp.float32),
    ],
    scratch_types=[
        pltpu.VMEM((CHD,), jnp.int32),
        pltpu.VMEM((CHD,), jnp.int32),
        pltpu.VMEM((CHD,), jnp.int32),
        pltpu.VMEM((2 * CHD,), jnp.int32),
        pltpu.VMEM((2 * CHD,), jnp.int32),
        pltpu.VMEM((2 * CHD, 128), jnp.float32),
        pltpu.VMEM((2 * CHD, 128), jnp.float32),
        pltpu.VMEM((CHD,), jnp.float32),
        pltpu.VMEM((CHD,), jnp.float32),
        pltpu.VMEM_SHARED((2 * ACC_ROWS, 128), jnp.float32),
        pltpu.VMEM_SHARED((ACC_ROWS,), jnp.float32),
        pltpu.SemaphoreType.DMA,
        pltpu.SemaphoreType.DMA,
        pltpu.SemaphoreType.DMA,
        pltpu.SemaphoreType.DMA,
    ],
)(_scatter_body)


def _final_body(acc_ref, s_ref, h_ref, o_ref):
    a = acc_ref[...].reshape(NT, D)
    o_ref[...] = a / (s_ref[...] + 1e-16) + h_ref[...]


def _finalize(acc_flat, s_flat, h):
    grid = (N // NT,)
    return pl.pallas_call(
        _final_body,
        grid=grid,
        in_specs=[
            pl.BlockSpec((2 * NT, 128), lambda i: (i, 0)),
            pl.BlockSpec((NT, 1), lambda i: (i, 0)),
            pl.BlockSpec((NT, D), lambda i: (i, 0)),
        ],
        out_specs=pl.BlockSpec((NT, D), lambda i: (i, 0)),
        out_shape=jax.ShapeDtypeStruct((N, D), jnp.float32),
    )(acc_flat, s_flat, h)


def kernel(n_feature, edge_index, attn_l, attn_r, few1, feb1, feg1, febt1,
           few2, feb2, feg2, febt2, fnw1, fnb1, fng1, fnbt1, fnw2, fnb2,
           fng2, fnbt2):
    rs = 1.0 / jnp.sqrt(jnp.float32(1.0 + 1e-5))

    w_pre = jnp.concatenate([
        attn_l.T, attn_r.T, few1[:D], few1[D:], fnw1[:D],
        jnp.zeros((D, 2), jnp.float32),
    ], axis=1)
    precomp, mx = _node_precompute(n_feature, w_pre)

    g = jnp.maximum(jnp.max(mx[:, 0, 0]) + jnp.max(mx[:, 0, 1]), 0.0)
    g = g.reshape(1, 1)

    src = edge_index[0]
    dst = edge_index[1]
    srcp = jnp.concatenate([src, jnp.zeros((EPAD - E,), jnp.int32)])
    dstp = jnp.concatenate([dst, jnp.zeros((EPAD - E,), jnp.int32)])
    dsts = jnp.concatenate([dst, jnp.full((EPAD - E,), 2 * NHALF, jnp.int32)])

    gs, gd = _edge_gather(precomp, srcp, dstp)

    c1 = (feg1 * rs).reshape(1, HID)
    b1 = (feg1 * rs * feb1 + febt1).reshape(1, HID)
    s2 = feg2 * rs
    w2 = few2 * s2[None, :]
    b2 = (s2 * feb2 + febt2).reshape(1, D)
    c3 = (fng1 * rs).reshape(1, HID)
    b3 = (fng1 * rs * fnb1 + fnbt1).reshape(1, HID)
    s4 = fng2 * rs
    w4 = fnw2 * s4[None, :]
    b4 = (s4 * fnb2 + fnbt2).reshape(1, D)

    w, e = _edge_dense(gs, gd, g, c1, b1, w2, b2, fnw1[D:], c3, b3, w4, b4)

    dst3 = dsts.reshape(16, D_CH, CHD)
    zrows = jnp.zeros((2 * ACC_ROWS, 128), jnp.float32)
    z1 = jnp.zeros((ACC_ROWS,), jnp.float32)
    acc, seg = _edge_scatter(w, e.reshape(EPAD), dst3, zrows, z1)

    acc_flat = acc.reshape(4 * NHALF, 128)
    s_flat = seg.reshape(2 * NHALF, 1)
    return _finalize(acc_flat, s_flat, n_feature)


# speedup vs baseline: 7.6561x; 1.1288x over previous
import functools

import jax
import jax.numpy as jnp
from jax import lax
from jax.experimental import pallas as pl
from jax.experimental.pallas import tpu as pltpu
from jax.experimental.pallas import tpu_sc as plsc

N = 10000
E = 160000
D = 256
HID = 4

NT = 400
ET = 1024
NW = 32
CHUNK = 128
EPAD = 163840
B_CH = EPAD // NW // CHUNK
CHD = 64
D_CH = EPAD // 16 // CHD
NHALF = 5120
ACC_ROWS = 5248
SLAB_I = ACC_ROWS // 16
SLAB_O = NHALF // 16


def _precomp_body(h_ref, w_ref, outs_ref, outd_ref, mx_ref):
    p = jnp.dot(h_ref[...], w_ref[...], preferred_element_type=jnp.float32)
    outs_ref[...] = p[:, :16]
    outd_ref[...] = p[:, 16:]
    mx_ref[...] = jnp.max(p, axis=0).reshape(1, 1, 32)


def _node_precompute(h, w_pre):
    grid = (N // NT,)
    return pl.pallas_call(
        _precomp_body,
        grid=grid,
        in_specs=[
            pl.BlockSpec((NT, D), lambda i: (i, 0)),
            pl.BlockSpec((D, 32), lambda i: (0, 0)),
        ],
        out_specs=[
            pl.BlockSpec((NT, 16), lambda i: (i, 0)),
            pl.BlockSpec((NT, 16), lambda i: (i, 0)),
            pl.BlockSpec((1, 1, 32), lambda i: (i, 0, 0)),
        ],
        out_shape=[
            jax.ShapeDtypeStruct((N, 16), jnp.float32),
            jax.ShapeDtypeStruct((N, 16), jnp.float32),
            jax.ShapeDtypeStruct((N // NT, 1, 32), jnp.float32),
        ],
    )(h, w_pre)


def _gather_body(pres_hbm, pred_hbm, src_hbm, dst_hbm, g_hbm,
                 is0, is1, id0, id1, rs0, rs1, rd0, rd1, su0, su1,
                 ss0, ss1, sd0, sd1):
    wid = lax.axis_index("s") * 2 + lax.axis_index("c")
    base = wid * (B_CH * CHUNK)
    isv = (is0, is1)
    idv = (id0, id1)
    rsv = (rs0, rs1)
    rdv = (rd0, rd1)
    suv = (su0, su1)
    ssv = (ss0, ss1)
    sdv = (sd0, sd1)

    pltpu.sync_copy(src_hbm.at[pl.ds(base, CHUNK)], is0)
    pltpu.sync_copy(dst_hbm.at[pl.ds(base, CHUNK)], id0)
    pltpu.async_copy(pres_hbm.at[is0], rs0, ss0)
    pltpu.async_copy(pred_hbm.at[id0], rd0, sd0)

    def pair(i, _):
        for b in (0, 1):
            j = 2 * i + b
            nxt = 1 - b
            @pl.when(j + 1 < B_CH)
            def _():
                noff = base + (j + 1) * CHUNK
                pltpu.sync_copy(src_hbm.at[pl.ds(noff, CHUNK)], isv[nxt])
                pltpu.sync_copy(dst_hbm.at[pl.ds(noff, CHUNK)], idv[nxt])
                pltpu.async_copy(pres_hbm.at[isv[nxt]], rsv[nxt], ssv[nxt])
                pltpu.async_copy(pred_hbm.at[idv[nxt]], rdv[nxt], sdv[nxt])
            off = base + j * CHUNK
            pltpu.make_async_copy(pres_hbm.at[isv[b]], rsv[b], ssv[b]).wait()
            pltpu.make_async_copy(pred_hbm.at[idv[b]], rdv[b], sdv[b]).wait()
            for r in range(CHUNK):
                suv[b][r, :] = rsv[b][r, :] + rdv[b][r, :]
            pltpu.sync_copy(suv[b], g_hbm.at[pl.ds(off, CHUNK)])
        return 0

    lax.fori_loop(0, B_CH // 2, pair, 0)


_edge_gather = functools.partial(
    pl.kernel,
    mesh=plsc.VectorSubcoreMesh(core_axis_name="c", subcore_axis_name="s"),
    compiler_params=pltpu.CompilerParams(use_tc_tiling_on_sc=False),
    out_type=jax.ShapeDtypeStruct((EPAD, 16), jnp.float32),
    scratch_types=[
        pltpu.VMEM((CHUNK,), jnp.int32),
        pltpu.VMEM((CHUNK,), jnp.int32),
        pltpu.VMEM((CHUNK,), jnp.int32),
        pltpu.VMEM((CHUNK,), jnp.int32),
        pltpu.VMEM((CHUNK, 16), jnp.float32),
        pltpu.VMEM((CHUNK, 16), jnp.float32),
        pltpu.VMEM((CHUNK, 16), jnp.float32),
        pltpu.VMEM((CHUNK, 16), jnp.float32),
        pltpu.VMEM((CHUNK, 16), jnp.float32),
        pltpu.VMEM((CHUNK, 16), jnp.float32),
        pltpu.SemaphoreType.DMA,
        pltpu.SemaphoreType.DMA,
        pltpu.SemaphoreType.DMA,
        pltpu.SemaphoreType.DMA,
    ],
)(_gather_body)


def _edge_body(g_ref2, g_ref, c1_ref, b1_ref, w2_ref, b2_ref,
               wn1_ref, c3_ref, b3_ref, w4_ref, b4_ref, w_ref, e_ref):
    gs = g_ref2[...]
    gd = gs
    u = gs[:, 0:1] + gd[:, 1:2]
    u = jnp.where(u >= 0, u, 0.2 * u)
    e = jnp.exp(u - g_ref[0, 0])
    x1 = gs[:, 2:6] + gd[:, 6:10]
    x1 = c1_ref[...] * x1 + b1_ref[...]
    x1 = jnp.where(x1 >= 0, x1, 0.01 * x1)
    x2 = jnp.dot(x1, w2_ref[...], preferred_element_type=jnp.float32)
    x2 = jnp.maximum(x2 + b2_ref[...], 0.0)
    t = gs[:, 10:14] + jnp.dot(x2, wn1_ref[...],
                               preferred_element_type=jnp.float32)
    x3 = c3_ref[...] * t + b3_ref[...]
    x3 = jnp.where(x3 >= 0, x3, 0.01 * x3)
    msg = jnp.dot(x3, w4_ref[...], preferred_element_type=jnp.float32)
    msg = jnp.maximum(msg + b4_ref[...], 0.0)
    w_ref[...] = (msg * e).reshape(2 * ET, 128)
    e_ref[...] = e


def _edge_dense(gv, g, c1, b1, w2, b2, wn1, c3, b3, w4, b4):
    grid = (EPAD // ET,)
    full = lambda r, c: pl.BlockSpec((r, c), lambda i: (0, 0))
    return pl.pallas_call(
        _edge_body,
        grid=grid,
        in_specs=[
            pl.BlockSpec((ET, 16), lambda i: (i, 0)),
            full(1, 1), full(1, HID), full(1, HID), full(HID, D), full(1, D),
            full(D, HID), full(1, HID), full(1, HID), full(HID, D), full(1, D),
        ],
        out_specs=[
            pl.BlockSpec((2 * ET, 128), lambda i: (i, 0)),
            pl.BlockSpec((ET, 1), lambda i: (i, 0)),
        ],
        out_shape=[
            jax.ShapeDtypeStruct((2 * EPAD, 128), jnp.float32),
            jax.ShapeDtypeStruct((EPAD, 1), jnp.float32),
        ],
    )(gv, g, c1, b1, w2, b2, wn1, c3, b3, w4, b4)


def _scatter_body(w_hbm, e_hbm, dst3_hbm, zrows_hbm, z1_hbm, acc_out, s_out,
                  idx_v, il0, il1, il2a, il2b, w0, w1, e0, e1, acc_sh, s_sh,
                  sw0, sw1, se0, se1):
    c = lax.axis_index("c")
    s = lax.axis_index("s")
    nbase = c * NHALF
    ilv = (il0, il1)
    il2v = (il2a, il2b)
    wv = (w0, w1)
    ev = (e0, e1)
    swv = (sw0, sw1)
    sev = (se0, se1)

    pltpu.sync_copy(zrows_hbm.at[pl.ds(s * (2 * SLAB_I), 2 * SLAB_I)],
                    acc_sh.at[pl.ds(s * (2 * SLAB_I), 2 * SLAB_I)])
    pltpu.sync_copy(z1_hbm.at[pl.ds(s * SLAB_I, SLAB_I)],
                    s_sh.at[pl.ds(s * SLAB_I, SLAB_I)])
    plsc.subcore_barrier()

    ebase = s * (D_CH * CHD)

    lane = lax.iota(jnp.int32, 16)

    def load_il(j, b):
        pltpu.sync_copy(dst3_hbm.at[s, j], idx_v)
        for k in range(CHD // 16):
            v = idx_v[pl.ds(k * 16, 16)]
            il = v - nbase
            bad = (il < 0) | (il >= NHALF)
            ilv[b][pl.ds(k * 16, 16)] = jnp.where(bad, NHALF, il)
        for t in range(2 * CHD // 16):
            src = plsc.load_gather(ilv[b], [t * 8 + (lane >> 1)])
            il2v[b][pl.ds(t * 16, 16)] = 2 * src + (lane & 1)

    pltpu.async_copy(w_hbm.at[pl.ds(2 * ebase, 2 * CHD)], w0, sw0)
    pltpu.async_copy(e_hbm.at[pl.ds(ebase, CHD)], e0, se0)
    load_il(0, 0)

    def pair(i, _):
        for b in (0, 1):
            j = 2 * i + b
            nxt = 1 - b
            @pl.when(j + 1 < D_CH)
            def _():
                noff = ebase + (j + 1) * CHD
                pltpu.async_copy(w_hbm.at[pl.ds(2 * noff, 2 * CHD)], wv[nxt],
                                 swv[nxt])
                pltpu.async_copy(e_hbm.at[pl.ds(noff, CHD)], ev[nxt],
                                 sev[nxt])
                load_il(j + 1, nxt)
            off = ebase + j * CHD
            pltpu.make_async_copy(w_hbm.at[pl.ds(2 * off, 2 * CHD)], wv[b],
                                  swv[b]).wait()
            pltpu.sync_copy(wv[b], acc_sh.at[il2v[b]], add=True)
            pltpu.make_async_copy(e_hbm.at[pl.ds(off, CHD)], ev[b],
                                  sev[b]).wait()
            pltpu.sync_copy(ev[b], s_sh.at[ilv[b]], add=True)
        return 0

    lax.fori_loop(0, D_CH // 2, pair, 0)
    plsc.subcore_barrier()
    pltpu.sync_copy(acc_sh.at[pl.ds(s * (2 * SLAB_O), 2 * SLAB_O)],
                    acc_out.at[c, pl.ds(s * (2 * SLAB_O), 2 * SLAB_O)])
    pltpu.sync_copy(s_sh.at[pl.ds(s * SLAB_O, SLAB_O)],
                    s_out.at[c, pl.ds(s * SLAB_O, SLAB_O)])


_edge_scatter = functools.partial(
    pl.kernel,
    mesh=plsc.VectorSubcoreMesh(core_axis_name="c", subcore_axis_name="s"),
    compiler_params=pltpu.CompilerParams(use_tc_tiling_on_sc=False,
                                         needs_layout_passes=False),
    out_type=[
        jax.ShapeDtypeStruct((2, 2 * NHALF, 128), jnp.float32),
        jax.ShapeDtypeStruct((2, NHALF), jnp.float32),
    ],
    scratch_types=[
        pltpu.VMEM((CHD,), jnp.int32),
        pltpu.VMEM((CHD,), jnp.int32),
        pltpu.VMEM((CHD,), jnp.int32),
        pltpu.VMEM((2 * CHD,), jnp.int32),
        pltpu.VMEM((2 * CHD,), jnp.int32),
        pltpu.VMEM((2 * CHD, 128), jnp.float32),
        pltpu.VMEM((2 * CHD, 128), jnp.float32),
        pltpu.VMEM((CHD,), jnp.float32),
        pltpu.VMEM((CHD,), jnp.float32),
        pltpu.VMEM_SHARED((2 * ACC_ROWS, 128), jnp.float32),
        pltpu.VMEM_SHARED((ACC_ROWS,), jnp.float32),
        pltpu.SemaphoreType.DMA,
        pltpu.SemaphoreType.DMA,
        pltpu.SemaphoreType.DMA,
        pltpu.SemaphoreType.DMA,
    ],
)(_scatter_body)


def _final_body(acc_ref, s_ref, h_ref, o_ref):
    a = acc_ref[...].reshape(NT, D)
    o_ref[...] = a / (s_ref[...] + 1e-16) + h_ref[...]


def _finalize(acc_flat, s_flat, h):
    grid = (N // NT,)
    return pl.pallas_call(
        _final_body,
        grid=grid,
        in_specs=[
            pl.BlockSpec((2 * NT, 128), lambda i: (i, 0)),
            pl.BlockSpec((NT, 1), lambda i: (i, 0)),
            pl.BlockSpec((NT, D), lambda i: (i, 0)),
        ],
        out_specs=pl.BlockSpec((NT, D), lambda i: (i, 0)),
        out_shape=jax.ShapeDtypeStruct((N, D), jnp.float32),
    )(acc_flat, s_flat, h)


def kernel(n_feature, edge_index, attn_l, attn_r, few1, feb1, feg1, febt1,
           few2, feb2, feg2, febt2, fnw1, fnb1, fng1, fnbt1, fnw2, fnb2,
           fng2, fnbt2):
    rs = 1.0 / jnp.sqrt(jnp.float32(1.0 + 1e-5))

    z1c = jnp.zeros((D, 1), jnp.float32)
    z4c = jnp.zeros((D, HID), jnp.float32)
    z2c = jnp.zeros((D, 2), jnp.float32)
    w_pre = jnp.concatenate([
        attn_l.T, z1c, few1[:D], z4c, fnw1[:D], z2c,
        z1c, attn_r.T, z4c, few1[D:], z4c, z2c,
    ], axis=1)
    tab_s, tab_d, mx = _node_precompute(n_feature, w_pre)

    g = jnp.maximum(jnp.max(mx[:, 0, 0]) + jnp.max(mx[:, 0, 17]), 0.0)
    g = g.reshape(1, 1)

    src = edge_index[0]
    dst = edge_index[1]
    srcp = jnp.concatenate([src, jnp.zeros((EPAD - E,), jnp.int32)])
    dstp = jnp.concatenate([dst, jnp.zeros((EPAD - E,), jnp.int32)])
    dsts = jnp.concatenate([dst, jnp.full((EPAD - E,), 2 * NHALF, jnp.int32)])

    gv = _edge_gather(tab_s, tab_d, srcp, dstp)

    c1 = (feg1 * rs).reshape(1, HID)
    b1 = (feg1 * rs * feb1 + febt1).reshape(1, HID)
    s2 = feg2 * rs
    w2 = few2 * s2[None, :]
    b2 = (s2 * feb2 + febt2).reshape(1, D)
    c3 = (fng1 * rs).reshape(1, HID)
    b3 = (fng1 * rs * fnb1 + fnbt1).reshape(1, HID)
    s4 = fng2 * rs
    w4 = fnw2 * s4[None, :]
    b4 = (s4 * fnb2 + fnbt2).reshape(1, D)

    w, e = _edge_dense(gv, g, c1, b1, w2, b2, fnw1[D:], c3, b3, w4, b4)

    dst3 = dsts.reshape(16, D_CH, CHD)
    zrows = jnp.zeros((2 * ACC_ROWS, 128), jnp.float32)
    z1 = jnp.zeros((ACC_ROWS,), jnp.float32)
    acc, seg = _edge_scatter(w, e.reshape(EPAD), dst3, zrows, z1)

    acc_flat = acc.reshape(4 * NHALF, 128)
    s_flat = seg.reshape(2 * NHALF, 1)
    return _finalize(acc_flat, s_flat, n_feature)
